# baseline (speedup 1.0000x reference)
import jax
import jax.numpy as jnp
from jax.experimental import pallas as pl
from jax.experimental.pallas import tpu as pltpu

_BLOCK_ROWS = 512


def _decode_body(x_ref, cb_ref, o_ref):
    x = x_ref[...]
    m = jnp.max(x, axis=-1, keepdims=True)
    e = jnp.exp(x - m)
    s = jnp.sum(e, axis=-1, keepdims=True)
    eb = e.astype(jnp.bfloat16)
    acc = jnp.dot(eb, cb_ref[...], preferred_element_type=jnp.float32)
    o_ref[...] = acc / s


def kernel(cls_logits, codebook):
    n, k = cls_logits.shape
    k2, d = codebook.shape
    assert k == k2
    br = _BLOCK_ROWS
    out = pl.pallas_call(
        _decode_body,
        grid=(n // br,),
        in_specs=[
            pl.BlockSpec((br, k), lambda i: (i, 0)),
            pl.BlockSpec((k, d), lambda i: (0, 0)),
        ],
        out_specs=pl.BlockSpec((br, d), lambda i: (i, 0)),
        out_shape=jax.ShapeDtypeStruct((n, d), jnp.float32),
        compiler_params=pltpu.CompilerParams(
            dimension_semantics=("arbitrary",),
        ),
    )(cls_logits, codebook.astype(jnp.bfloat16))
    return out


# speedup vs baseline: 1.1055x; 1.1055x over previous
import jax
import jax.numpy as jnp
from jax.experimental import pallas as pl
from jax.experimental.pallas import tpu as pltpu

_BLOCK_ROWS = 512


def _decode_body(x_ref, cb_ref, o_ref):
    x = x_ref[...]
    m = jnp.max(x, axis=-1, keepdims=True)
    eb = jnp.exp(x - m).astype(jnp.bfloat16)
    acc = jnp.dot(eb, cb_ref[...], preferred_element_type=jnp.float32)
    d = o_ref.shape[-1]
    o_ref[...] = acc[:, :d] / acc[:, d:d + 1]


def kernel(cls_logits, codebook):
    n, k = cls_logits.shape
    k2, d = codebook.shape
    assert k == k2
    br = _BLOCK_ROWS
    cb_aug = jnp.concatenate(
        [codebook, jnp.ones((k, 1), codebook.dtype)], axis=1
    ).astype(jnp.bfloat16)
    out = pl.pallas_call(
        _decode_body,
        grid=(n // br,),
        in_specs=[
            pl.BlockSpec((br, k), lambda i: (i, 0)),
            pl.BlockSpec((k, d + 1), lambda i: (0, 0)),
        ],
        out_specs=pl.BlockSpec((br, d), lambda i: (i, 0)),
        out_shape=jax.ShapeDtypeStruct((n, d), jnp.float32),
        compiler_params=pltpu.CompilerParams(
            dimension_semantics=("arbitrary",),
        ),
    )(cls_logits, cb_aug)
    return out
